# baseline (speedup 1.0000x reference)
import functools

import jax
import jax.numpy as jnp
from jax import lax
from jax.experimental import pallas as pl
from jax.experimental.pallas import tpu as pltpu
from jax.experimental.pallas import tpu_sc as plsc

N = 10000
E = 320000
D = 128
EPS = 1e-5

NC = 2
NS = 16
NW = NC * NS
HALF = E // 2
EP = HALF // NW
CH = 40
NCH = EP // CH
MR = CH
NMR = EP // MR
N2 = 10240
RPT = N2 // NS
LANES = 16

_mesh = plsc.VectorSubcoreMesh(core_axis_name="c", subcore_axis_name="s")


@functools.partial(
    pl.kernel,
    mesh=_mesh,
    out_type=jax.ShapeDtypeStruct((HALF, D), jnp.float32),
    scratch_types=[
        pltpu.VMEM((NCH, CH), jnp.int32),
        pltpu.VMEM((NCH, CH), jnp.int32),
        pltpu.VMEM((2, CH, D), jnp.float32),
        pltpu.VMEM((2, CH, D), jnp.float32),
        pltpu.SemaphoreType.DMA,
        pltpu.SemaphoreType.DMA,
        pltpu.SemaphoreType.DMA,
        pltpu.SemaphoreType.DMA,
        pltpu.SemaphoreType.DMA,
        pltpu.SemaphoreType.DMA,
    ],
)
def _sc_gather_sum(xs_hbm, xr_hbm, eidx_hbm, g_hbm,
                   sidx_v, ridx_v, buf_a, buf_b,
                   ga0, ga1, gb0, gb1, w0, w1):
    wid = lax.axis_index("s") * NC + lax.axis_index("c")
    base = wid * EP
    pltpu.sync_copy(eidx_hbm.at[0, wid], sidx_v)
    pltpu.sync_copy(eidx_hbm.at[1, wid], ridx_v)

    def fire(j, slot, sa, sb):
        pltpu.async_copy(xs_hbm.at[sidx_v.at[j]], buf_a.at[slot], sa)
        pltpu.async_copy(xr_hbm.at[ridx_v.at[j]], buf_b.at[slot], sb)

    def wait_gather(slot, sa, sb):
        pltpu.make_async_copy(xs_hbm.at[pl.ds(0, CH)], buf_a.at[slot], sa).wait()
        pltpu.make_async_copy(xs_hbm.at[pl.ds(0, CH)], buf_b.at[slot], sb).wait()

    def wait_write(slot, sw):
        pltpu.make_async_copy(buf_a.at[slot], g_hbm.at[pl.ds(base, CH)], sw).wait()

    def add_rows(slot):
        def add_row(r, c2):
            for cc in range(D // LANES):
                sl = pl.ds(cc * LANES, LANES)
                buf_a[slot, r, sl] = buf_a[slot, r, sl] + buf_b[slot, r, sl]
            return c2
        lax.fori_loop(0, CH, add_row, 0)

    fire(0, 0, ga0, gb0)

    def body(i, carry):
        j0 = 2 * i
        j1 = j0 + 1
        j2 = j0 + 2

        @pl.when(i > 0)
        def _():
            wait_write(1, w1)

        fire(j1, 1, ga1, gb1)
        wait_gather(0, ga0, gb0)
        add_rows(0)
        pltpu.async_copy(buf_a.at[0], g_hbm.at[pl.ds(base + j0 * CH, CH)], w0)
        wait_write(0, w0)
        fire(j2, 0, ga0, gb0)
        wait_gather(1, ga1, gb1)
        add_rows(1)
        pltpu.async_copy(buf_a.at[1], g_hbm.at[pl.ds(base + j1 * CH, CH)], w1)
        return carry

    lax.fori_loop(0, (NCH - 1) // 2, body, 0)
    wait_gather(0, ga0, gb0)
    add_rows(0)
    pltpu.sync_copy(buf_a.at[0], g_hbm.at[pl.ds(base + (NCH - 1) * CH, CH)])
    wait_write(1, w1)


@functools.partial(
    pl.kernel,
    mesh=_mesh,
    out_type=jax.ShapeDtypeStruct((NC, N2, D), jnp.float32),
    scratch_types=[
        pltpu.VMEM((NCH, CH), jnp.int32),
        pltpu.VMEM((2, MR, D), jnp.float32),
        pltpu.VMEM_SHARED((N2, D), jnp.float32),
        pltpu.SemaphoreType.DMA,
        pltpu.SemaphoreType.DMA,
    ],
)
def _sc_scatter_add(msg_hbm, eidx_hbm, zero_hbm, out_hbm, ridx_v, mbuf, aggsh,
                    r0, r1):
    c = lax.axis_index("c")
    s = lax.axis_index("s")
    wid = s * NC + c
    base = wid * EP
    pltpu.sync_copy(zero_hbm, aggsh.at[pl.ds(s * RPT, RPT)])
    pltpu.sync_copy(eidx_hbm.at[1, wid], ridx_v)
    plsc.subcore_barrier()

    def fire(j, slot, sem):
        pltpu.async_copy(msg_hbm.at[pl.ds(base + j * MR, MR)], mbuf.at[slot], sem)

    def wait_read(slot, sem):
        pltpu.make_async_copy(msg_hbm.at[pl.ds(base, MR)], mbuf.at[slot], sem).wait()

    fire(0, 0, r0)

    def outer(i, carry):
        j0 = 2 * i
        j1 = j0 + 1
        j2 = j0 + 2
        fire(j1, 1, r1)
        wait_read(0, r0)
        pltpu.sync_copy(mbuf.at[0], aggsh.at[ridx_v.at[j0]], add=True)
        fire(j2, 0, r0)
        wait_read(1, r1)
        pltpu.sync_copy(mbuf.at[1], aggsh.at[ridx_v.at[j1]], add=True)
        return carry

    lax.fori_loop(0, (NMR - 1) // 2, outer, 0)
    wait_read(0, r0)
    pltpu.sync_copy(mbuf.at[0], aggsh.at[ridx_v.at[NMR - 1]], add=True)
    plsc.subcore_barrier()
    pltpu.sync_copy(aggsh.at[pl.ds(s * RPT, RPT)],
                    out_hbm.at[c, pl.ds(s * RPT, RPT)])


def _tc_precompute(x_ref, w1a_ref, w1b_ref, xs_ref, xr_ref):
    x = x_ref[...]
    xs_ref[...] = jnp.dot(x, w1a_ref[...], preferred_element_type=jnp.float32)
    xr_ref[...] = jnp.dot(x, w1b_ref[...], preferred_element_type=jnp.float32)


def _tc_edge_mlp(g_ref, ea_ref, w1e_ref, b1_ref, w2_ref, b2_ref,
                 w3_ref, b3_ref, gam_ref, bet_ref, eo_ref, msg_ref):
    ea = ea_ref[...]
    h = g_ref[...] + jnp.dot(ea, w1e_ref[...],
                             preferred_element_type=jnp.float32) + b1_ref[...]
    h = h * jax.nn.sigmoid(h)
    h = jnp.dot(h, w2_ref[...], preferred_element_type=jnp.float32) + b2_ref[...]
    h = h * jax.nn.sigmoid(h)
    h = jnp.dot(h, w3_ref[...], preferred_element_type=jnp.float32) + b3_ref[...]
    mu = jnp.mean(h, axis=-1, keepdims=True)
    d = h - mu
    var = jnp.mean(d * d, axis=-1, keepdims=True)
    msg = d * lax.rsqrt(var + EPS) * gam_ref[...] + bet_ref[...]
    msg_ref[...] = msg
    eo_ref[...] = ea + msg


def _tc_node_mlp(x_ref, a0_ref, a1_ref, a2_ref, a3_ref, w1a_ref, w1b_ref,
                 b1_ref, w2_ref, b2_ref, w3_ref, b3_ref, gam_ref, bet_ref,
                 xo_ref):
    x = x_ref[...]
    agg = (a0_ref[0] + a1_ref[0]) + (a2_ref[0] + a3_ref[0])
    h = (jnp.dot(x, w1a_ref[...], preferred_element_type=jnp.float32)
         + jnp.dot(agg, w1b_ref[...], preferred_element_type=jnp.float32)
         + b1_ref[...])
    h = h * jax.nn.sigmoid(h)
    h = jnp.dot(h, w2_ref[...], preferred_element_type=jnp.float32) + b2_ref[...]
    h = h * jax.nn.sigmoid(h)
    h = jnp.dot(h, w3_ref[...], preferred_element_type=jnp.float32) + b3_ref[...]
    mu = jnp.mean(h, axis=-1, keepdims=True)
    d = h - mu
    var = jnp.mean(d * d, axis=-1, keepdims=True)
    xo_ref[...] = x + d * lax.rsqrt(var + EPS) * gam_ref[...] + bet_ref[...]


_BN = 2000
_BE = 4000

_w_spec = pl.BlockSpec((D, D), lambda i: (0, 0))
_v_spec = pl.BlockSpec((1, D), lambda i: (0, 0))


_NBLK = E // _BE
_HBLK = _NBLK // 2


def _edge_mlp_half(g_h, edge_attr, ew, blk0, eo_prev, *wargs):
    in_specs = [pl.BlockSpec((_BE, D), lambda i: (i, 0)),
                pl.BlockSpec((_BE, D), lambda i, b=blk0: (i + b, 0)),
                _w_spec, _v_spec, _w_spec, _v_spec, _w_spec, _v_spec,
                _v_spec, _v_spec]
    args = [g_h, edge_attr, ew] + list(wargs)
    aliases = {}
    if eo_prev is not None:
        in_specs.append(pl.BlockSpec(memory_space=pltpu.MemorySpace.HBM))
        args.append(eo_prev)
        aliases = {10: 0}

    def body(*refs):
        if eo_prev is not None:
            refs = refs[:10] + refs[11:]
        _tc_edge_mlp(*refs)

    return pl.pallas_call(
        body,
        grid=(_HBLK,),
        in_specs=in_specs,
        out_specs=[pl.BlockSpec((_BE, D), lambda i, b=blk0: (i + b, 0)),
                   pl.BlockSpec((_BE, D), lambda i: (i, 0))],
        out_shape=[jax.ShapeDtypeStruct((E, D), jnp.float32),
                   jax.ShapeDtypeStruct((HALF, D), jnp.float32)],
        input_output_aliases=aliases,
    )(*args)


def kernel(x, edge_attr, edge_index,
           eW1, eb1, eW2, eb2, eW3, eb3, eg, ebeta,
           nW1, nb1, nW2, nb2, nW3, nb3, ng, nbeta):
    eidx = edge_index.astype(jnp.int32).reshape(2, 2, NW, NCH, CH)
    eidx_a = eidx[:, 0]
    eidx_b = eidx[:, 1]

    xs, xr = pl.pallas_call(
        _tc_precompute,
        grid=(N // _BN,),
        in_specs=[pl.BlockSpec((_BN, D), lambda i: (i, 0)), _w_spec, _w_spec],
        out_specs=[pl.BlockSpec((_BN, D), lambda i: (i, 0))] * 2,
        out_shape=[jax.ShapeDtypeStruct((N, D), jnp.float32)] * 2,
    )(x, eW1[:D], eW1[D:2 * D])

    g_a = _sc_gather_sum(xs, xr, eidx_a)
    g_b = _sc_gather_sum(xs, xr, eidx_b)

    wargs = (eb1.reshape(1, D), eW2, eb2.reshape(1, D), eW3, eb3.reshape(1, D),
             eg.reshape(1, D), ebeta.reshape(1, D))
    eo_a, msg_a = _edge_mlp_half(g_a, edge_attr, eW1[2 * D:], 0, None, *wargs)
    eo, msg_b = _edge_mlp_half(g_b, edge_attr, eW1[2 * D:], _HBLK, eo_a, *wargs)

    zero = jnp.zeros((RPT, D), jnp.float32)
    agg_a = _sc_scatter_add(msg_a, eidx_a, zero)
    agg_b = _sc_scatter_add(msg_b, eidx_b, zero)

    x_out = pl.pallas_call(
        _tc_node_mlp,
        grid=(N // _BN,),
        in_specs=[pl.BlockSpec((_BN, D), lambda i: (i, 0)),
                  pl.BlockSpec((1, _BN, D), lambda i: (0, i, 0)),
                  pl.BlockSpec((1, _BN, D), lambda i: (1, i, 0)),
                  pl.BlockSpec((1, _BN, D), lambda i: (0, i, 0)),
                  pl.BlockSpec((1, _BN, D), lambda i: (1, i, 0)),
                  _w_spec, _w_spec, _v_spec, _w_spec, _v_spec, _w_spec,
                  _v_spec, _v_spec, _v_spec],
        out_specs=pl.BlockSpec((_BN, D), lambda i: (i, 0)),
        out_shape=jax.ShapeDtypeStruct((N, D), jnp.float32),
    )(x, agg_a, agg_a, agg_b, agg_b, nW1[:D], nW1[D:], nb1.reshape(1, D),
      nW2, nb2.reshape(1, D), nW3, nb3.reshape(1, D), ng.reshape(1, D),
      nbeta.reshape(1, D))

    return (x_out, eo)


# speedup vs baseline: 1.1071x; 1.1071x over previous
import functools

import jax
import jax.numpy as jnp
from jax import lax
from jax.experimental import pallas as pl
from jax.experimental.pallas import tpu as pltpu
from jax.experimental.pallas import tpu_sc as plsc

N = 10000
E = 320000
D = 128
EPS = 1e-5

NC = 2
NS = 16
NW = NC * NS
N2 = 10240
RPT = N2 // NS
LANES = 16

_mesh = plsc.VectorSubcoreMesh(core_axis_name="c", subcore_axis_name="s")


def _make_gather(ep, ch):
    nch = ep // ch
    seg = ep * NW

    @functools.partial(
        pl.kernel,
        mesh=_mesh,
        out_type=jax.ShapeDtypeStruct((seg, D), jnp.float32),
        scratch_types=[
            pltpu.VMEM((nch, ch), jnp.int32),
            pltpu.VMEM((nch, ch), jnp.int32),
            pltpu.VMEM((2, ch, D), jnp.float32),
            pltpu.VMEM((2, ch, D), jnp.float32),
            pltpu.SemaphoreType.DMA,
            pltpu.SemaphoreType.DMA,
            pltpu.SemaphoreType.DMA,
            pltpu.SemaphoreType.DMA,
            pltpu.SemaphoreType.DMA,
            pltpu.SemaphoreType.DMA,
        ],
    )
    def gather_sum(xs_hbm, xr_hbm, eidx_hbm, g_hbm,
                   sidx_v, ridx_v, buf_a, buf_b,
                   ga0, ga1, gb0, gb1, w0, w1):
        wid = lax.axis_index("s") * NC + lax.axis_index("c")
        base = wid * ep
        pltpu.sync_copy(eidx_hbm.at[0, wid], sidx_v)
        pltpu.sync_copy(eidx_hbm.at[1, wid], ridx_v)

        def fire(j, slot, sa, sb):
            pltpu.async_copy(xs_hbm.at[sidx_v.at[j]], buf_a.at[slot], sa)
            pltpu.async_copy(xr_hbm.at[ridx_v.at[j]], buf_b.at[slot], sb)

        def wait_gather(slot, sa, sb):
            pltpu.make_async_copy(xs_hbm.at[pl.ds(0, ch)], buf_a.at[slot], sa).wait()
            pltpu.make_async_copy(xs_hbm.at[pl.ds(0, ch)], buf_b.at[slot], sb).wait()

        def wait_write(slot, sw):
            pltpu.make_async_copy(buf_a.at[slot], g_hbm.at[pl.ds(base, ch)], sw).wait()

        def add_rows(slot):
            def add_row(r, c2):
                for cc in range(D // LANES):
                    sl = pl.ds(cc * LANES, LANES)
                    buf_a[slot, r, sl] = buf_a[slot, r, sl] + buf_b[slot, r, sl]
                return c2
            lax.fori_loop(0, ch, add_row, 0)

        fire(0, 0, ga0, gb0)

        def body(i, carry):
            j0 = 2 * i
            j1 = j0 + 1
            j2 = j0 + 2

            @pl.when(i > 0)
            def _():
                wait_write(1, w1)

            fire(j1, 1, ga1, gb1)
            wait_gather(0, ga0, gb0)
            add_rows(0)
            pltpu.async_copy(buf_a.at[0], g_hbm.at[pl.ds(base + j0 * ch, ch)], w0)
            wait_write(0, w0)
            fire(j2, 0, ga0, gb0)
            wait_gather(1, ga1, gb1)
            add_rows(1)
            pltpu.async_copy(buf_a.at[1], g_hbm.at[pl.ds(base + j1 * ch, ch)], w1)
            return carry

        lax.fori_loop(0, (nch - 1) // 2, body, 0)
        wait_gather(0, ga0, gb0)
        add_rows(0)
        pltpu.sync_copy(buf_a.at[0], g_hbm.at[pl.ds(base + (nch - 1) * ch, ch)])
        wait_write(1, w1)

    return gather_sum


def _make_scatter(ep, ch):
    nch = ep // ch

    @functools.partial(
        pl.kernel,
        mesh=_mesh,
        out_type=jax.ShapeDtypeStruct((NC, N2, D), jnp.float32),
        scratch_types=[
            pltpu.VMEM((nch, ch), jnp.int32),
            pltpu.VMEM((2, ch, D), jnp.float32),
            pltpu.VMEM_SHARED((N2, D), jnp.float32),
            pltpu.SemaphoreType.DMA,
            pltpu.SemaphoreType.DMA,
        ],
    )
    def scatter_add(msg_hbm, eidx_hbm, zero_hbm, out_hbm, ridx_v, mbuf, aggsh,
                    r0, r1):
        c = lax.axis_index("c")
        s = lax.axis_index("s")
        wid = s * NC + c
        base = wid * ep
        pltpu.sync_copy(zero_hbm, aggsh.at[pl.ds(s * RPT, RPT)])
        pltpu.sync_copy(eidx_hbm.at[1, wid], ridx_v)
        plsc.subcore_barrier()

        def fire(j, slot, sem):
            pltpu.async_copy(msg_hbm.at[pl.ds(base + j * ch, ch)], mbuf.at[slot], sem)

        def wait_read(slot, sem):
            pltpu.make_async_copy(msg_hbm.at[pl.ds(base, ch)], mbuf.at[slot], sem).wait()

        fire(0, 0, r0)

        def outer(i, carry):
            j0 = 2 * i
            j1 = j0 + 1
            j2 = j0 + 2
            fire(j1, 1, r1)
            wait_read(0, r0)
            pltpu.sync_copy(mbuf.at[0], aggsh.at[ridx_v.at[j0]], add=True)
            fire(j2, 0, r0)
            wait_read(1, r1)
            pltpu.sync_copy(mbuf.at[1], aggsh.at[ridx_v.at[j1]], add=True)
            return carry

        lax.fori_loop(0, (nch - 1) // 2, outer, 0)
        wait_read(0, r0)
        pltpu.sync_copy(mbuf.at[0], aggsh.at[ridx_v.at[nch - 1]], add=True)
        plsc.subcore_barrier()
        pltpu.sync_copy(aggsh.at[pl.ds(s * RPT, RPT)],
                        out_hbm.at[c, pl.ds(s * RPT, RPT)])

    return scatter_add


_gather_a = _make_gather(3000, 120)
_gather_b = _make_gather(4000, 80)
_scatter_a = _make_scatter(3000, 120)
_scatter_b = _make_scatter(4000, 80)


def _tc_precompute(x_ref, w1a_ref, w1b_ref, xs_ref, xr_ref):
    x = x_ref[...]
    xs_ref[...] = jnp.dot(x, w1a_ref[...], preferred_element_type=jnp.float32)
    xr_ref[...] = jnp.dot(x, w1b_ref[...], preferred_element_type=jnp.float32)


def _tc_edge_mlp(g_ref, ea_ref, w1e_ref, b1_ref, w2_ref, b2_ref,
                 w3_ref, b3_ref, gam_ref, bet_ref, eo_ref, msg_ref):
    ea = ea_ref[...]
    h = g_ref[...] + jnp.dot(ea, w1e_ref[...],
                             preferred_element_type=jnp.float32) + b1_ref[...]
    h = h * jax.nn.sigmoid(h)
    h = jnp.dot(h, w2_ref[...], preferred_element_type=jnp.float32) + b2_ref[...]
    h = h * jax.nn.sigmoid(h)
    h = jnp.dot(h, w3_ref[...], preferred_element_type=jnp.float32) + b3_ref[...]
    mu = jnp.mean(h, axis=-1, keepdims=True)
    d = h - mu
    var = jnp.mean(d * d, axis=-1, keepdims=True)
    msg = d * lax.rsqrt(var + EPS) * gam_ref[...] + bet_ref[...]
    msg_ref[...] = msg
    eo_ref[...] = ea + msg


def _tc_node_mlp(x_ref, a0_ref, a1_ref, a2_ref, a3_ref, a4_ref, a5_ref,
                 w1a_ref, w1b_ref, b1_ref, w2_ref, b2_ref, w3_ref, b3_ref,
                 gam_ref, bet_ref, xo_ref):
    x = x_ref[...]
    agg = ((a0_ref[0] + a1_ref[0]) + (a2_ref[0] + a3_ref[0])
           + (a4_ref[0] + a5_ref[0]))
    h = (jnp.dot(x, w1a_ref[...], preferred_element_type=jnp.float32)
         + jnp.dot(agg, w1b_ref[...], preferred_element_type=jnp.float32)
         + b1_ref[...])
    h = h * jax.nn.sigmoid(h)
    h = jnp.dot(h, w2_ref[...], preferred_element_type=jnp.float32) + b2_ref[...]
    h = h * jax.nn.sigmoid(h)
    h = jnp.dot(h, w3_ref[...], preferred_element_type=jnp.float32) + b3_ref[...]
    mu = jnp.mean(h, axis=-1, keepdims=True)
    d = h - mu
    var = jnp.mean(d * d, axis=-1, keepdims=True)
    xo_ref[...] = x + d * lax.rsqrt(var + EPS) * gam_ref[...] + bet_ref[...]


_BN = 2000
_BE = 4000

_w_spec = pl.BlockSpec((D, D), lambda i: (0, 0))
_v_spec = pl.BlockSpec((1, D), lambda i: (0, 0))


_SEG_EDGES = (96000, 128000, 96000)
_SEG_BLK0 = (0, 24, 56)


def _edge_mlp_seg(g_h, edge_attr, ew, blk0, eo_prev, *wargs):
    nblk = g_h.shape[0] // _BE
    in_specs = [pl.BlockSpec((_BE, D), lambda i: (i, 0)),
                pl.BlockSpec((_BE, D), lambda i, b=blk0: (i + b, 0)),
                _w_spec, _v_spec, _w_spec, _v_spec, _w_spec, _v_spec,
                _v_spec, _v_spec]
    args = [g_h, edge_attr, ew] + list(wargs)
    aliases = {}
    if eo_prev is not None:
        in_specs.append(pl.BlockSpec(memory_space=pltpu.MemorySpace.HBM))
        args.append(eo_prev)
        aliases = {10: 0}

    def body(*refs):
        if eo_prev is not None:
            refs = refs[:10] + refs[11:]
        _tc_edge_mlp(*refs)

    return pl.pallas_call(
        body,
        grid=(nblk,),
        in_specs=in_specs,
        out_specs=[pl.BlockSpec((_BE, D), lambda i, b=blk0: (i + b, 0)),
                   pl.BlockSpec((_BE, D), lambda i: (i, 0))],
        out_shape=[jax.ShapeDtypeStruct((E, D), jnp.float32),
                   jax.ShapeDtypeStruct((g_h.shape[0], D), jnp.float32)],
        input_output_aliases=aliases,
    )(*args)


def kernel(x, edge_attr, edge_index,
           eW1, eb1, eW2, eb2, eW3, eb3, eg, ebeta,
           nW1, nb1, nW2, nb2, nW3, nb3, ng, nbeta):
    eidx32 = edge_index.astype(jnp.int32)
    e0, e1, e2 = _SEG_EDGES
    eidx_a = eidx32[:, :e0].reshape(2, NW, 3000 // 120, 120)
    eidx_b = eidx32[:, e0:e0 + e1].reshape(2, NW, 4000 // 80, 80)
    eidx_c = eidx32[:, e0 + e1:].reshape(2, NW, 3000 // 120, 120)

    xs, xr = pl.pallas_call(
        _tc_precompute,
        grid=(N // _BN,),
        in_specs=[pl.BlockSpec((_BN, D), lambda i: (i, 0)), _w_spec, _w_spec],
        out_specs=[pl.BlockSpec((_BN, D), lambda i: (i, 0))] * 2,
        out_shape=[jax.ShapeDtypeStruct((N, D), jnp.float32)] * 2,
    )(x, eW1[:D], eW1[D:2 * D])

    g_a = _gather_a(xs, xr, eidx_a)
    g_b = _gather_b(xs, xr, eidx_b)
    g_c = _gather_a(xs, xr, eidx_c)

    wargs = (eb1.reshape(1, D), eW2, eb2.reshape(1, D), eW3, eb3.reshape(1, D),
             eg.reshape(1, D), ebeta.reshape(1, D))
    ew = eW1[2 * D:]
    eo_a, msg_a = _edge_mlp_seg(g_a, edge_attr, ew, _SEG_BLK0[0], None, *wargs)
    eo_b, msg_b = _edge_mlp_seg(g_b, edge_attr, ew, _SEG_BLK0[1], eo_a, *wargs)
    eo, msg_c = _edge_mlp_seg(g_c, edge_attr, ew, _SEG_BLK0[2], eo_b, *wargs)

    zero = jnp.zeros((RPT, D), jnp.float32)
    agg_a = _scatter_a(msg_a, eidx_a, zero)
    agg_b = _scatter_b(msg_b, eidx_b, zero)
    agg_c = _scatter_a(msg_c, eidx_c, zero)

    x_out = pl.pallas_call(
        _tc_node_mlp,
        grid=(N // _BN,),
        in_specs=[pl.BlockSpec((_BN, D), lambda i: (i, 0)),
                  pl.BlockSpec((1, _BN, D), lambda i: (0, i, 0)),
                  pl.BlockSpec((1, _BN, D), lambda i: (1, i, 0)),
                  pl.BlockSpec((1, _BN, D), lambda i: (0, i, 0)),
                  pl.BlockSpec((1, _BN, D), lambda i: (1, i, 0)),
                  pl.BlockSpec((1, _BN, D), lambda i: (0, i, 0)),
                  pl.BlockSpec((1, _BN, D), lambda i: (1, i, 0)),
                  _w_spec, _w_spec, _v_spec, _w_spec, _v_spec, _w_spec,
                  _v_spec, _v_spec, _v_spec],
        out_specs=pl.BlockSpec((_BN, D), lambda i: (i, 0)),
        out_shape=jax.ShapeDtypeStruct((N, D), jnp.float32),
    )(x, agg_a, agg_a, agg_b, agg_b, agg_c, agg_c, nW1[:D], nW1[D:],
      nb1.reshape(1, D), nW2, nb2.reshape(1, D), nW3, nb3.reshape(1, D),
      ng.reshape(1, D), nbeta.reshape(1, D))

    return (x_out, eo)
